# baseline (speedup 1.0000x reference)
import functools

import jax
import jax.numpy as jnp
from jax import lax
from jax.experimental import pallas as pl
from jax.experimental.pallas import tpu as pltpu
from jax.experimental.pallas import tpu_sc as plsc

NC = 2
NS = 16
LANES = 16
NW = NC * NS
R = 3


@functools.lru_cache(maxsize=None)
def _build(B, S, D):
    C = S // NW
    T = 8
    NT = C // T
    ND = D // LANES
    assert S % NW == 0 and C % T == 0 and D % LANES == 0
    assert NT % R == 2 and NT > R

    mesh = plsc.VectorSubcoreMesh(
        core_axis_name="c", subcore_axis_name="s",
        num_cores=NC, num_subcores=NS)

    scratch = [pltpu.VMEM((T, D), jnp.float32) for _ in range(R)]
    scratch += [pltpu.VMEM((B, T, D), jnp.float32) for _ in range(R)]
    scratch += [pltpu.SemaphoreType.DMA for _ in range(3 * R)]

    @functools.partial(
        pl.kernel,
        out_type=jax.ShapeDtypeStruct((B, S, D), jnp.float32),
        mesh=mesh,
        scratch_types=scratch,
    )
    def k(x_hbm, tab_hbm, out_hbm, *bufs):
        tbufs = bufs[0:R]
        xbufs = bufs[R:2 * R]
        sem_t = bufs[2 * R:3 * R]
        sem_x = bufs[3 * R:4 * R]
        sem_s = bufs[4 * R:5 * R]

        cid = lax.axis_index("c")
        sid = lax.axis_index("s")
        wid = sid * NC + cid
        base = wid * C

        def load_tab(t, q):
            pltpu.async_copy(tab_hbm.at[pl.ds(base + t * T, T)],
                             tbufs[q], sem_t[q])

        def load_x(t, q):
            pltpu.async_copy(x_hbm.at[:, pl.ds(base + t * T, T)],
                             xbufs[q], sem_x[q])

        def wait(src, dst, sem):
            pltpu.make_async_copy(src, dst, sem).wait()

        def wait_store(t, q):
            wait(xbufs[q], out_hbm.at[:, pl.ds(base + t * T, T)], sem_s[q])

        for t0 in range(2):
            load_tab(t0, t0)
            load_x(t0, t0)

        def tile(t, q, may_be_first, reload_):
            p = base + t * T
            tb = tbufs[q]
            xb = xbufs[q]
            qn = (q + 2) % R

            if reload_:
                load_tab(t + 2, qn)
            wait(tab_hbm.at[pl.ds(p, T)], tb, sem_t[q])
            wait(x_hbm.at[:, pl.ds(p, T)], xb, sem_x[q])

            def row_body(r, c2):
                for j in range(ND):
                    sl = pl.ds(j * LANES, LANES)
                    tv = tb[r, sl]
                    for b in range(B):
                        plsc.addupdate(xb.at[b, r, sl], tv)
                return c2

            lax.fori_loop(0, T, row_body, 0, unroll=False)

            pltpu.async_copy(xb, out_hbm.at[:, pl.ds(p, T)], sem_s[q])

            if reload_:
                def reload(t=t, qn=qn):
                    wait_store(t - 1, qn)
                    load_x(t + 2, qn)

                if may_be_first:
                    @pl.when(t > 0)
                    def _():
                        reload()

                    @pl.when(t == 0)
                    def _(t=t, qn=qn):
                        load_x(t + 2, qn)
                else:
                    reload()

        def round_body(i, carry):
            t = R * i
            tile(t, 0, True, True)
            tile(t + 1, 1, False, True)
            tile(t + 2, 2, False, True)
            return carry

        lax.fori_loop(0, NT // R, round_body, 0, unroll=False)
        tile(NT - 2, (NT - 2) % R, False, False)
        tile(NT - 1, (NT - 1) % R, False, False)

        for t0 in range(NT - R, NT):
            wait_store(t0, t0 % R)

    return k


def kernel(x, pos_table):
    B, S, D = x.shape
    return _build(B, S, D)(x, pos_table[:S])
